# initial kernel scaffold (unmeasured)
import functools

import jax
import jax.numpy as jnp
from jax import lax
from jax.experimental import pallas as pl
from jax.experimental.pallas import tpu as pltpu

N_Z = 4
E_LOC = 4
T_LOC = 256
T = N_Z * T_LOC
D = 1024
F = 2048
E = N_Z * E_LOC

_ANY = pltpu.ANY if hasattr(pltpu, "ANY") else pltpu.MemorySpace.ANY
_CompilerParams = getattr(pltpu, "CompilerParams", None) or getattr(
    pltpu, "TPUCompilerParams"
)


def kernel(x, router, W1, W2):
    def body(
        x_ref, r_ref, w1_hbm, w2_hbm, out_ref,
        xg_ref, rg_ref, gg_ref, acc_ref, rs_ref,
        w1_buf, w2_buf,
        xg_send, xg_recv, rg_send, rg_recv,
        gg_send, gg_recv, rs_send, rs_recv,
        wdma_sem,
    ):
        my_x = lax.axis_index("x")
        my_y = lax.axis_index("y")
        my_z = lax.axis_index("z")

        def peer(d):
            return (my_x, my_y, lax.rem(my_z + d, N_Z))

        def rdma(src, dst, ssem, rsem, dev):
            return pltpu.make_async_remote_copy(
                src_ref=src, dst_ref=dst, send_sem=ssem, recv_sem=rsem,
                device_id=dev, device_id_type=pl.DeviceIdType.MESH,
            )

        w1_dma = pltpu.make_async_copy(w1_hbm.at[0], w1_buf, wdma_sem.at[0])
        w2_dma = pltpu.make_async_copy(w2_hbm.at[0], w2_buf, wdma_sem.at[1])
        w1_dma.start()
        w2_dma.start()

        bsem = pltpu.get_barrier_semaphore()
        for d in range(1, N_Z):
            pl.semaphore_signal(
                bsem, inc=1, device_id=peer(d),
                device_id_type=pl.DeviceIdType.MESH,
            )
        pl.semaphore_wait(bsem, N_Z - 1)

        rg_ref[0] = r_ref[...]
        xg_ref[0] = x_ref[...].astype(jnp.bfloat16)
        sends = []
        for d in range(1, N_Z):
            r_s = rdma(rg_ref.at[0], rg_ref.at[N_Z - d],
                       rg_send.at[d], rg_recv.at[N_Z - d], peer(d))
            x_s = rdma(xg_ref.at[0], xg_ref.at[N_Z - d],
                       xg_send.at[d], xg_recv.at[N_Z - d], peer(d))
            r_s.start()
            x_s.start()
            sends += [r_s, x_s]

        for d in range(1, N_Z):
            rdma(rg_ref.at[0], rg_ref.at[d],
                 rg_send.at[d], rg_recv.at[d], peer(d)).wait_recv()
        r_rot = jnp.concatenate([rg_ref[d] for d in range(N_Z)], axis=1)
        g0 = jnp.dot(
            x_ref[...], r_rot,
            preferred_element_type=jnp.float32,
            precision=lax.Precision.HIGHEST,
        )
        gg_ref[0] = g0
        for d in range(1, N_Z):
            g_s = rdma(gg_ref.at[0], gg_ref.at[N_Z - d],
                       gg_send.at[d], gg_recv.at[N_Z - d], peer(d))
            g_s.start()
            sends.append(g_s)

        for d in range(1, N_Z):
            rdma(gg_ref.at[0], gg_ref.at[d],
                 gg_send.at[d], gg_recv.at[d], peer(d)).wait_recv()
        g_all = jnp.concatenate([gg_ref[d] for d in range(N_Z)], axis=0)
        col = lax.broadcasted_iota(jnp.int32, (T, E), 1)
        v1 = jnp.max(g_all, axis=1, keepdims=True)
        i1 = jnp.min(jnp.where(g_all == v1, col, E), axis=1, keepdims=True)
        g_m = jnp.where(col == i1, jnp.float32(-1e30), g_all)
        v2 = jnp.max(g_m, axis=1, keepdims=True)
        i2 = jnp.min(jnp.where(g_m == v2, col, E), axis=1, keepdims=True)
        wt1 = 1.0 / (1.0 + jnp.exp(v2 - v1))
        wt2 = 1.0 - wt1
        i1g = E_LOC * lax.rem(my_z + i1 // E_LOC, N_Z) + lax.rem(i1, E_LOC)
        i2g = E_LOC * lax.rem(my_z + i2 // E_LOC, N_Z) + lax.rem(i2, E_LOC)

        for d in range(1, N_Z):
            rdma(xg_ref.at[0], xg_ref.at[d],
                 xg_send.at[d], xg_recv.at[d], peer(d)).wait_recv()
        x_all = xg_ref[...].reshape(T, D)

        for j in range(E_LOC):
            if j == 0:
                w1_dma.wait()
                w2_dma.wait()
            else:
                d1 = pltpu.make_async_copy(w1_hbm.at[j], w1_buf, wdma_sem.at[0])
                d2 = pltpu.make_async_copy(w2_hbm.at[j], w2_buf, wdma_sem.at[1])
                d1.start()
                d2.start()
                d1.wait()
                d2.wait()
            w1b = w1_buf[...].astype(jnp.bfloat16)
            h = jnp.maximum(
                jnp.dot(x_all, w1b, preferred_element_type=jnp.float32), 0.0
            )
            w2b = w2_buf[...].astype(jnp.bfloat16)
            y = jnp.dot(
                h.astype(jnp.bfloat16), w2b,
                preferred_element_type=jnp.float32,
            )
            e_glob = my_z * E_LOC + j
            c = jnp.where(i1g == e_glob, wt1, 0.0) + jnp.where(
                i2g == e_glob, wt2, 0.0
            )
            contrib = (c * y).reshape(N_Z, T_LOC, D)
            if j == 0:
                acc_ref[...] = contrib
            else:
                acc_ref[...] = acc_ref[...] + contrib

        for d in range(1, N_Z):
            rs_s = rdma(acc_ref.at[d], rs_ref.at[N_Z - d],
                        rs_send.at[d], rs_recv.at[N_Z - d], peer(d))
            rs_s.start()
            sends.append(rs_s)
        for d in range(1, N_Z):
            rdma(acc_ref.at[d], rs_ref.at[d],
                 rs_send.at[d], rs_recv.at[d], peer(d)).wait_recv()
        out_ref[...] = acc_ref[0] + rs_ref[1] + rs_ref[2] + rs_ref[3]

        for s in sends:
            s.wait_send()

        @functools.partial(
            pl.run_scoped, exit_sem=pltpu.SemaphoreType.REGULAR
        )
        def _(exit_sem):
            for d in range(1, N_Z):
                pl.semaphore_signal(
                    exit_sem, inc=1, device_id=peer(d),
                    device_id_type=pl.DeviceIdType.MESH,
                )
            pl.semaphore_wait(exit_sem, N_Z - 1)

    return pl.pallas_call(
        body,
        out_shape=jax.ShapeDtypeStruct((T_LOC, D), jnp.float32),
        in_specs=[
            pl.BlockSpec(memory_space=pltpu.VMEM),
            pl.BlockSpec(memory_space=pltpu.VMEM),
            pl.BlockSpec(memory_space=_ANY),
            pl.BlockSpec(memory_space=_ANY),
        ],
        out_specs=pl.BlockSpec(memory_space=pltpu.VMEM),
        scratch_shapes=[
            pltpu.VMEM((N_Z, T_LOC, D), jnp.bfloat16),
            pltpu.VMEM((N_Z, D, E_LOC), jnp.float32),
            pltpu.VMEM((N_Z, T_LOC, E), jnp.float32),
            pltpu.VMEM((N_Z, T_LOC, D), jnp.float32),
            pltpu.VMEM((N_Z, T_LOC, D), jnp.float32),
            pltpu.VMEM((D, F), jnp.float32),
            pltpu.VMEM((F, D), jnp.float32),
            pltpu.SemaphoreType.DMA((N_Z,)),
            pltpu.SemaphoreType.DMA((N_Z,)),
            pltpu.SemaphoreType.DMA((N_Z,)),
            pltpu.SemaphoreType.DMA((N_Z,)),
            pltpu.SemaphoreType.DMA((N_Z,)),
            pltpu.SemaphoreType.DMA((N_Z,)),
            pltpu.SemaphoreType.DMA((N_Z,)),
            pltpu.SemaphoreType.DMA((N_Z,)),
            pltpu.SemaphoreType.DMA((2,)),
        ],
        compiler_params=_CompilerParams(collective_id=0),
    )(x, router, W1, W2)


# baseline (device time: 168740 ns/iter reference)
import functools

import jax
import jax.numpy as jnp
from jax import lax
from jax.experimental import pallas as pl
from jax.experimental.pallas import tpu as pltpu

N_Z = 4
E_LOC = 4
T_LOC = 256
T = N_Z * T_LOC
D = 1024
F = 2048
E = N_Z * E_LOC

_ANY = pltpu.MemorySpace.HBM
_CompilerParams = getattr(pltpu, "CompilerParams", None) or getattr(
    pltpu, "TPUCompilerParams"
)


def kernel(x, router, W1, W2):
    def body(
        x_ref, r_ref, w1_hbm, w2_hbm, out_ref,
        xg_ref, rg_ref, gg_ref, acc_ref, rs_ref,
        w1_buf, w2_buf,
        xg_send, xg_recv, rg_send, rg_recv,
        gg_send, gg_recv, rs_send, rs_recv,
        wdma_sem,
    ):
        my_x = lax.axis_index("x")
        my_y = lax.axis_index("y")
        my_z = lax.axis_index("z")

        def peer(d):
            return (my_x, my_y, lax.rem(my_z + d, N_Z))

        def rdma(src, dst, ssem, rsem, dev):
            return pltpu.make_async_remote_copy(
                src_ref=src, dst_ref=dst, send_sem=ssem, recv_sem=rsem,
                device_id=dev, device_id_type=pl.DeviceIdType.MESH,
            )

        w1_dma = pltpu.make_async_copy(w1_hbm.at[0], w1_buf, wdma_sem.at[0])
        w2_dma = pltpu.make_async_copy(w2_hbm.at[0], w2_buf, wdma_sem.at[1])
        w1_dma.start()
        w2_dma.start()

        bsem = pltpu.get_barrier_semaphore()
        for d in range(1, N_Z):
            pl.semaphore_signal(
                bsem, inc=1, device_id=peer(d),
                device_id_type=pl.DeviceIdType.MESH,
            )
        pl.semaphore_wait(bsem, N_Z - 1)

        rg_ref[0] = r_ref[...]
        xg_ref[0] = x_ref[...].astype(jnp.bfloat16)
        sends = []
        for d in range(1, N_Z):
            r_s = rdma(rg_ref.at[0], rg_ref.at[N_Z - d],
                       rg_send.at[d], rg_recv.at[N_Z - d], peer(d))
            x_s = rdma(xg_ref.at[0], xg_ref.at[N_Z - d],
                       xg_send.at[d], xg_recv.at[N_Z - d], peer(d))
            r_s.start()
            x_s.start()
            sends += [r_s, x_s]

        for d in range(1, N_Z):
            rdma(rg_ref.at[0], rg_ref.at[d],
                 rg_send.at[d], rg_recv.at[d], peer(d)).wait_recv()
        r_rot = jnp.concatenate([rg_ref[d] for d in range(N_Z)], axis=1)
        r_glob = pltpu.roll(r_rot, E_LOC * my_z, 1)
        g0 = jnp.dot(
            x_ref[...], r_glob,
            preferred_element_type=jnp.float32,
            precision=lax.Precision.HIGHEST,
        )
        gg_ref[0] = g0
        for d in range(1, N_Z):
            g_s = rdma(gg_ref.at[0], gg_ref.at[N_Z - d],
                       gg_send.at[d], gg_recv.at[N_Z - d], peer(d))
            g_s.start()
            sends.append(g_s)

        for d in range(1, N_Z):
            rdma(gg_ref.at[0], gg_ref.at[d],
                 gg_send.at[d], gg_recv.at[d], peer(d)).wait_recv()
        g_all = jnp.concatenate([gg_ref[d] for d in range(N_Z)], axis=0)
        col = lax.broadcasted_iota(jnp.int32, (T, E), 1)
        v1 = jnp.max(g_all, axis=1, keepdims=True)
        i1 = jnp.min(jnp.where(g_all == v1, col, E), axis=1, keepdims=True)
        g_m = jnp.where(col == i1, jnp.float32(-1e30), g_all)
        v2 = jnp.max(g_m, axis=1, keepdims=True)
        i2 = jnp.min(jnp.where(g_m == v2, col, E), axis=1, keepdims=True)
        wt1 = 1.0 / (1.0 + jnp.exp(v2 - v1))
        wt2 = 1.0 - wt1
        i1g, i2g = i1, i2

        for d in range(1, N_Z):
            rdma(xg_ref.at[0], xg_ref.at[d],
                 xg_send.at[d], xg_recv.at[d], peer(d)).wait_recv()
        x_all = xg_ref[...].reshape(T, D)

        for j in range(E_LOC):
            if j == 0:
                w1_dma.wait()
                w2_dma.wait()
            else:
                d1 = pltpu.make_async_copy(w1_hbm.at[j], w1_buf, wdma_sem.at[0])
                d2 = pltpu.make_async_copy(w2_hbm.at[j], w2_buf, wdma_sem.at[1])
                d1.start()
                d2.start()
                d1.wait()
                d2.wait()
            w1b = w1_buf[...].astype(jnp.bfloat16)
            h = jnp.maximum(
                jnp.dot(x_all, w1b, preferred_element_type=jnp.float32), 0.0
            )
            w2b = w2_buf[...].astype(jnp.bfloat16)
            y = jnp.dot(
                h.astype(jnp.bfloat16), w2b,
                preferred_element_type=jnp.float32,
            )
            e_glob = my_z * E_LOC + j
            c = jnp.where(i1g == e_glob, wt1, 0.0) + jnp.where(
                i2g == e_glob, wt2, 0.0
            )
            contrib = (c * y).reshape(N_Z, T_LOC, D)
            if j == 0:
                acc_ref[...] = contrib
            else:
                acc_ref[...] = acc_ref[...] + contrib

        for d in range(1, N_Z):
            rs_s = rdma(acc_ref.at[d], rs_ref.at[N_Z - d],
                        rs_send.at[d], rs_recv.at[N_Z - d], peer(d))
            rs_s.start()
            sends.append(rs_s)
        for d in range(1, N_Z):
            rdma(acc_ref.at[d], rs_ref.at[d],
                 rs_send.at[d], rs_recv.at[d], peer(d)).wait_recv()
        out_ref[...] = acc_ref[0] + rs_ref[1] + rs_ref[2] + rs_ref[3]

        for s in sends:
            s.wait_send()

        @functools.partial(
            pl.run_scoped, exit_sem=pltpu.SemaphoreType.REGULAR
        )
        def _(exit_sem):
            for d in range(1, N_Z):
                pl.semaphore_signal(
                    exit_sem, inc=1, device_id=peer(d),
                    device_id_type=pl.DeviceIdType.MESH,
                )
            pl.semaphore_wait(exit_sem, N_Z - 1)

    return pl.pallas_call(
        body,
        out_shape=jax.ShapeDtypeStruct((T_LOC, D), jnp.float32),
        in_specs=[
            pl.BlockSpec(memory_space=pltpu.VMEM),
            pl.BlockSpec(memory_space=pltpu.VMEM),
            pl.BlockSpec(memory_space=_ANY),
            pl.BlockSpec(memory_space=_ANY),
        ],
        out_specs=pl.BlockSpec(memory_space=pltpu.VMEM),
        scratch_shapes=[
            pltpu.VMEM((N_Z, T_LOC, D), jnp.bfloat16),
            pltpu.VMEM((N_Z, D, E_LOC), jnp.float32),
            pltpu.VMEM((N_Z, T_LOC, E), jnp.float32),
            pltpu.VMEM((N_Z, T_LOC, D), jnp.float32),
            pltpu.VMEM((N_Z, T_LOC, D), jnp.float32),
            pltpu.VMEM((D, F), jnp.float32),
            pltpu.VMEM((F, D), jnp.float32),
            pltpu.SemaphoreType.DMA((N_Z,)),
            pltpu.SemaphoreType.DMA((N_Z,)),
            pltpu.SemaphoreType.DMA((N_Z,)),
            pltpu.SemaphoreType.DMA((N_Z,)),
            pltpu.SemaphoreType.DMA((N_Z,)),
            pltpu.SemaphoreType.DMA((N_Z,)),
            pltpu.SemaphoreType.DMA((N_Z,)),
            pltpu.SemaphoreType.DMA((N_Z,)),
            pltpu.SemaphoreType.DMA((2,)),
        ],
        compiler_params=_CompilerParams(
            collective_id=0, vmem_limit_bytes=60 * 1024 * 1024
        ),
    )(x, router, W1, W2)


# device time: 128829 ns/iter; 1.3098x vs baseline; 1.3098x over previous
import functools

import jax
import jax.numpy as jnp
from jax import lax
from jax.experimental import pallas as pl
from jax.experimental.pallas import tpu as pltpu

N_Z = 4
E_LOC = 4
T_LOC = 256
T = N_Z * T_LOC
D = 1024
F = 2048
E = N_Z * E_LOC

_ANY = pltpu.MemorySpace.HBM
_CompilerParams = getattr(pltpu, "CompilerParams", None) or getattr(
    pltpu, "TPUCompilerParams"
)


def kernel(x, router, W1, W2):
    def body(
        x_ref, r_ref, w1_hbm, w2_hbm, out_ref,
        xg_ref, rg_ref, gg_ref, acc_ref, rs_ref, rsb_ref,
        w1_buf, w2_buf,
        xg_send, xg_recv, rg_send, rg_recv,
        gg_send, gg_recv, rs_send, rs_recv,
        wdma_sem,
    ):
        my_x = lax.axis_index("x")
        my_y = lax.axis_index("y")
        my_z = lax.axis_index("z")

        def peer(d):
            return (my_x, my_y, lax.rem(my_z + d, N_Z))

        def rdma(src, dst, ssem, rsem, dev):
            return pltpu.make_async_remote_copy(
                src_ref=src, dst_ref=dst, send_sem=ssem, recv_sem=rsem,
                device_id=dev, device_id_type=pl.DeviceIdType.MESH,
            )

        w_dmas = {}

        def start_wdma(j):
            slot = j % 2
            d1 = pltpu.make_async_copy(
                w1_hbm.at[j], w1_buf.at[slot], wdma_sem.at[slot, 0]
            )
            d2 = pltpu.make_async_copy(
                w2_hbm.at[j], w2_buf.at[slot], wdma_sem.at[slot, 1]
            )
            d1.start()
            d2.start()
            w_dmas[j] = (d1, d2)

        start_wdma(0)

        bsem = pltpu.get_barrier_semaphore()
        for d in range(1, N_Z):
            pl.semaphore_signal(
                bsem, inc=1, device_id=peer(d),
                device_id_type=pl.DeviceIdType.MESH,
            )
        pl.semaphore_wait(bsem, N_Z - 1)

        rg_ref[0] = r_ref[...]
        xg_ref[0] = x_ref[...].astype(jnp.bfloat16)
        sends = []
        for d in range(1, N_Z):
            r_s = rdma(rg_ref.at[0], rg_ref.at[N_Z - d],
                       rg_send.at[d], rg_recv.at[N_Z - d], peer(d))
            x_s = rdma(xg_ref.at[0], xg_ref.at[N_Z - d],
                       xg_send.at[d], xg_recv.at[N_Z - d], peer(d))
            r_s.start()
            x_s.start()
            sends += [r_s, x_s]

        for d in range(1, N_Z):
            rdma(rg_ref.at[0], rg_ref.at[d],
                 rg_send.at[d], rg_recv.at[d], peer(d)).wait_recv()
        r_rot = jnp.concatenate([rg_ref[d] for d in range(N_Z)], axis=1)
        r_glob = pltpu.roll(r_rot, E_LOC * my_z, 1)
        g0 = jnp.dot(
            x_ref[...], r_glob,
            preferred_element_type=jnp.float32,
            precision=lax.Precision.HIGHEST,
        )
        gg_ref[0] = g0
        for d in range(1, N_Z):
            g_s = rdma(gg_ref.at[0], gg_ref.at[N_Z - d],
                       gg_send.at[d], gg_recv.at[N_Z - d], peer(d))
            g_s.start()
            sends.append(g_s)

        for d in range(1, N_Z):
            rdma(gg_ref.at[0], gg_ref.at[d],
                 gg_send.at[d], gg_recv.at[d], peer(d)).wait_recv()
        g_all = jnp.concatenate([gg_ref[d] for d in range(N_Z)], axis=0)
        col = lax.broadcasted_iota(jnp.int32, (T, E), 1)
        v1 = jnp.max(g_all, axis=1, keepdims=True)
        i1 = jnp.min(jnp.where(g_all == v1, col, E), axis=1, keepdims=True)
        g_m = jnp.where(col == i1, jnp.float32(-1e30), g_all)
        v2 = jnp.max(g_m, axis=1, keepdims=True)
        i2 = jnp.min(jnp.where(g_m == v2, col, E), axis=1, keepdims=True)
        wt1 = 1.0 / (1.0 + jnp.exp(v2 - v1))
        wt2 = 1.0 - wt1
        i1g, i2g = i1, i2

        for d in range(1, N_Z):
            rdma(xg_ref.at[0], xg_ref.at[d],
                 xg_send.at[d], xg_recv.at[d], peer(d)).wait_recv()
        x_all = xg_ref[...].reshape(T, D)

        for j in range(E_LOC):
            d1, d2 = w_dmas[j]
            d1.wait()
            d2.wait()
            if j + 1 < E_LOC:
                start_wdma(j + 1)
            slot = j % 2
            w1b = w1_buf[slot].astype(jnp.bfloat16)
            h = jnp.maximum(
                jnp.dot(x_all, w1b, preferred_element_type=jnp.float32), 0.0
            )
            w2b = w2_buf[slot].astype(jnp.bfloat16)
            y = jnp.dot(
                h.astype(jnp.bfloat16), w2b,
                preferred_element_type=jnp.float32,
            )
            e_glob = my_z * E_LOC + j
            c = jnp.where(i1g == e_glob, wt1, 0.0) + jnp.where(
                i2g == e_glob, wt2, 0.0
            )
            contrib = (c * y).reshape(N_Z, T_LOC, D)
            if j == 0:
                acc_ref[...] = contrib
            else:
                acc_ref[...] = acc_ref[...] + contrib

        rsb_ref[...] = acc_ref[...].astype(jnp.bfloat16)
        for d in range(1, N_Z):
            rs_s = rdma(rsb_ref.at[d], rs_ref.at[N_Z - d],
                        rs_send.at[d], rs_recv.at[N_Z - d], peer(d))
            rs_s.start()
            sends.append(rs_s)
        for d in range(1, N_Z):
            rdma(rsb_ref.at[d], rs_ref.at[d],
                 rs_send.at[d], rs_recv.at[d], peer(d)).wait_recv()
        out_ref[...] = acc_ref[0] + (
            rs_ref[1].astype(jnp.float32)
            + rs_ref[2].astype(jnp.float32)
            + rs_ref[3].astype(jnp.float32)
        )

        for s in sends:
            s.wait_send()

        @functools.partial(
            pl.run_scoped, exit_sem=pltpu.SemaphoreType.REGULAR
        )
        def _(exit_sem):
            for d in range(1, N_Z):
                pl.semaphore_signal(
                    exit_sem, inc=1, device_id=peer(d),
                    device_id_type=pl.DeviceIdType.MESH,
                )
            pl.semaphore_wait(exit_sem, N_Z - 1)

    return pl.pallas_call(
        body,
        out_shape=jax.ShapeDtypeStruct((T_LOC, D), jnp.float32),
        in_specs=[
            pl.BlockSpec(memory_space=pltpu.VMEM),
            pl.BlockSpec(memory_space=pltpu.VMEM),
            pl.BlockSpec(memory_space=_ANY),
            pl.BlockSpec(memory_space=_ANY),
        ],
        out_specs=pl.BlockSpec(memory_space=pltpu.VMEM),
        scratch_shapes=[
            pltpu.VMEM((N_Z, T_LOC, D), jnp.bfloat16),
            pltpu.VMEM((N_Z, D, E_LOC), jnp.float32),
            pltpu.VMEM((N_Z, T_LOC, E), jnp.float32),
            pltpu.VMEM((N_Z, T_LOC, D), jnp.float32),
            pltpu.VMEM((N_Z, T_LOC, D), jnp.bfloat16),
            pltpu.VMEM((N_Z, T_LOC, D), jnp.bfloat16),
            pltpu.VMEM((2, D, F), jnp.float32),
            pltpu.VMEM((2, F, D), jnp.float32),
            pltpu.SemaphoreType.DMA((N_Z,)),
            pltpu.SemaphoreType.DMA((N_Z,)),
            pltpu.SemaphoreType.DMA((N_Z,)),
            pltpu.SemaphoreType.DMA((N_Z,)),
            pltpu.SemaphoreType.DMA((N_Z,)),
            pltpu.SemaphoreType.DMA((N_Z,)),
            pltpu.SemaphoreType.DMA((N_Z,)),
            pltpu.SemaphoreType.DMA((N_Z,)),
            pltpu.SemaphoreType.DMA((2, 2)),
        ],
        compiler_params=_CompilerParams(
            collective_id=0, vmem_limit_bytes=60 * 1024 * 1024
        ),
    )(x, router, W1, W2)


# device time: 90944 ns/iter; 1.8554x vs baseline; 1.4166x over previous
import functools
import os

import jax
import jax.numpy as jnp
from jax import lax
from jax.experimental import pallas as pl
from jax.experimental.pallas import tpu as pltpu

N_Z = 4
E_LOC = 4
T_LOC = 256
T = N_Z * T_LOC
D = 1024
F = 2048
E = N_Z * E_LOC

_ANY = pltpu.MemorySpace.HBM
_ABL = os.environ.get("ABLATE", "")
_CompilerParams = getattr(pltpu, "CompilerParams", None) or getattr(
    pltpu, "TPUCompilerParams"
)


def kernel(x, router, W1, W2):
    def body(
        x_ref, r_ref, w1_hbm, w2_hbm, out_ref,
        rg_ref, xb_ref,
        psrc_ref, pr_ref, prx_ref, sxy_ref, pry_ref,
        stage_ref, w1c_ref, w2c_ref,
        rg_send, rg_recv, xp_send, xp_recv,
        pp_send, pp_recv, ax_send, ax_recv, ay_send, ay_recv,
        wdma_sem,
    ):
        my_x = lax.axis_index("x")
        my_y = lax.axis_index("y")
        my_z = lax.axis_index("z")
        q = 2 * my_x + my_y

        def zpeer(d):
            return (my_x, my_y, lax.rem(my_z + d, N_Z))

        fwd = zpeer(q)
        bwd = zpeer(N_Z - q)
        xpartner = (1 - my_x, my_y, my_z)
        ypartner = (my_x, 1 - my_y, my_z)

        def rdma(src, dst, ssem, rsem, dev):
            return pltpu.make_async_remote_copy(
                src_ref=src, dst_ref=dst, send_sem=ssem, recv_sem=rsem,
                device_id=dev, device_id_type=pl.DeviceIdType.MESH,
            )

        pieces = []
        for j in range(E_LOC):
            for kind in ("w1", "w2"):
                for h in range(2):
                    pieces.append((kind, j, h))

        def piece_dma(i, slot):
            kind, j, h = pieces[i]
            if kind == "w1":
                src = w1_hbm.at[j, :, pl.ds(h * 1024, 1024)]
            else:
                src = w2_hbm.at[j, pl.ds(h * 1024, 1024), :]
            return pltpu.make_async_copy(
                src, stage_ref.at[slot], wdma_sem.at[slot]
            )

        def piece_store(i, slot):
            kind, j, h = pieces[i]
            val = stage_ref[slot].astype(jnp.bfloat16)
            if kind == "w1":
                w1c_ref[j, :, pl.ds(h * 1024, 1024)] = val
            else:
                w2c_ref[j, pl.ds(h * 1024, 1024), :] = val

        do_ffn = _ABL not in (
            "no_ffn", "comm_only", "bare", "x_only", "rg_only", "ar_only"
        )
        do_ar = _ABL not in ("no_ar", "comm_only", "bare", "x_only", "rg_only")
        do_comm = _ABL != "bare"
        do_xp = _ABL not in ("rg_only", "ar_only")
        do_rg = _ABL not in ("x_only", "ar_only")
        do_pp = _ABL != "ar_only"

        bsem = pltpu.get_barrier_semaphore()
        for dev in [zpeer(1), zpeer(2), zpeer(3), xpartner, ypartner]:
            pl.semaphore_signal(
                bsem, inc=1, device_id=dev,
                device_id_type=pl.DeviceIdType.MESH,
            )
        pl.semaphore_wait(bsem, 5)

        if not do_comm:
            out_ref[...] = x_ref[...]

            @functools.partial(
                pl.run_scoped, exit_sem0=pltpu.SemaphoreType.REGULAR
            )
            def _(exit_sem0):
                for dev in [zpeer(1), zpeer(2), zpeer(3), xpartner, ypartner]:
                    pl.semaphore_signal(
                        exit_sem0, inc=1, device_id=dev,
                        device_id_type=pl.DeviceIdType.MESH,
                    )
                pl.semaphore_wait(exit_sem0, 5)

            return

        rg_ref[0] = r_ref[...]
        for d in range(1, N_Z) if do_rg else []:
            rdma(rg_ref.at[0], rg_ref.at[N_Z - d],
                 rg_send.at[d], rg_recv.at[N_Z - d], zpeer(d)).start()

        if do_xp:
            @pl.when(q != 0)
            def _():
                rdma(x_ref, xb_ref, xp_send, xp_recv, bwd).start()

            @pl.when(q == 0)
            def _():
                xb_ref[...] = x_ref[...]
        else:
            xb_ref[...] = x_ref[...]

        if do_ffn:
            dmas = {}
            for i in range(2):
                dmas[i] = piece_dma(i, i)
                dmas[i].start()
            for i in range(len(pieces)):
                slot = i % 2
                dmas[i].wait()
                piece_store(i, slot)
                if i + 2 < len(pieces):
                    dmas[i + 2] = piece_dma(i + 2, slot)
                    dmas[i + 2].start()

        for d in range(1, N_Z) if do_rg else []:
            rdma(rg_ref.at[0], rg_ref.at[d],
                 rg_send.at[d], rg_recv.at[d], zpeer(d)).wait_recv()
        r_rot = jnp.concatenate([rg_ref[d] for d in range(N_Z)], axis=1)
        r_glob = pltpu.roll(r_rot, E_LOC * my_z, 1)

        if do_xp:
            @pl.when(q != 0)
            def _():
                rdma(x_ref, xb_ref, xp_send, xp_recv, fwd).wait_recv()
        xb_f32 = xb_ref[...]
        g = jnp.dot(
            xb_f32, r_glob,
            preferred_element_type=jnp.float32,
            precision=lax.Precision.HIGHEST,
        )
        col = lax.broadcasted_iota(jnp.int32, (T_LOC, E), 1)
        v1 = jnp.max(g, axis=1, keepdims=True)
        i1 = jnp.min(jnp.where(g == v1, col, E), axis=1, keepdims=True)
        g_m = jnp.where(col == i1, jnp.float32(-1e30), g)
        v2 = jnp.max(g_m, axis=1, keepdims=True)
        i2 = jnp.min(jnp.where(g_m == v2, col, E), axis=1, keepdims=True)
        wt1 = 1.0 / (1.0 + jnp.exp(v2 - v1))
        wt2 = 1.0 - wt1

        xb = xb_f32.astype(jnp.bfloat16)
        p_acc = None
        for j in range(E_LOC if do_ffn else 0):
            h = jnp.maximum(
                jnp.dot(xb, w1c_ref[j], preferred_element_type=jnp.float32),
                0.0,
            )
            y = jnp.dot(
                h.astype(jnp.bfloat16), w2c_ref[j],
                preferred_element_type=jnp.float32,
            )
            e_glob = my_z * E_LOC + j
            c = jnp.where(i1 == e_glob, wt1, 0.0) + jnp.where(
                i2 == e_glob, wt2, 0.0
            )
            p_acc = c * y if p_acc is None else p_acc + c * y
        psrc_ref[...] = xb if p_acc is None else p_acc.astype(jnp.bfloat16)

        if do_ar:
            if do_pp:
                @pl.when(q != 0)
                def _():
                    rdma(psrc_ref, pr_ref, pp_send, pp_recv, fwd).start()

                @pl.when(q == 0)
                def _():
                    pr_ref[...] = psrc_ref[...]

                @pl.when(q != 0)
                def _():
                    rdma(psrc_ref, pr_ref, pp_send, pp_recv, bwd).wait_recv()
            else:
                pr_ref[...] = psrc_ref[...]

            ax = rdma(pr_ref, prx_ref, ax_send, ax_recv, xpartner)
            ax.start()
            ax.wait_recv()
            s = pr_ref[...].astype(jnp.float32) + prx_ref[...].astype(
                jnp.float32
            )
            sxy_ref[...] = s.astype(jnp.bfloat16)
            ay = rdma(sxy_ref, pry_ref, ay_send, ay_recv, ypartner)
            ay.start()
            ay.wait_recv()
            out_ref[...] = s + pry_ref[...].astype(jnp.float32)
        else:
            out_ref[...] = psrc_ref[...].astype(jnp.float32)

        for d in range(1, N_Z) if do_rg else []:
            rdma(rg_ref.at[0], rg_ref.at[N_Z - d],
                 rg_send.at[d], rg_recv.at[N_Z - d], zpeer(d)).wait_send()

        if do_xp:
            @pl.when(q != 0)
            def _():
                rdma(x_ref, xb_ref, xp_send, xp_recv, bwd).wait_send()

        if do_ar:
            if do_pp:
                @pl.when(q != 0)
                def _():
                    rdma(psrc_ref, pr_ref, pp_send, pp_recv, fwd).wait_send()

            ax.wait_send()
            ay.wait_send()

        @functools.partial(
            pl.run_scoped, exit_sem=pltpu.SemaphoreType.REGULAR
        )
        def _(exit_sem):
            for dev in [zpeer(1), zpeer(2), zpeer(3), xpartner, ypartner]:
                pl.semaphore_signal(
                    exit_sem, inc=1, device_id=dev,
                    device_id_type=pl.DeviceIdType.MESH,
                )
            pl.semaphore_wait(exit_sem, 5)

    return pl.pallas_call(
        body,
        out_shape=jax.ShapeDtypeStruct((T_LOC, D), jnp.float32),
        in_specs=[
            pl.BlockSpec(memory_space=pltpu.VMEM),
            pl.BlockSpec(memory_space=pltpu.VMEM),
            pl.BlockSpec(memory_space=_ANY),
            pl.BlockSpec(memory_space=_ANY),
        ],
        out_specs=pl.BlockSpec(memory_space=pltpu.VMEM),
        scratch_shapes=[
            pltpu.VMEM((N_Z, D, E_LOC), jnp.float32),
            pltpu.VMEM((T_LOC, D), jnp.float32),
            pltpu.VMEM((T_LOC, D), jnp.bfloat16),
            pltpu.VMEM((T_LOC, D), jnp.bfloat16),
            pltpu.VMEM((T_LOC, D), jnp.bfloat16),
            pltpu.VMEM((T_LOC, D), jnp.bfloat16),
            pltpu.VMEM((T_LOC, D), jnp.bfloat16),
            pltpu.VMEM((2, 1024, 1024), jnp.float32),
            pltpu.VMEM((E_LOC, D, F), jnp.bfloat16),
            pltpu.VMEM((E_LOC, F, D), jnp.bfloat16),
            pltpu.SemaphoreType.DMA((N_Z,)),
            pltpu.SemaphoreType.DMA((N_Z,)),
            pltpu.SemaphoreType.DMA,
            pltpu.SemaphoreType.DMA,
            pltpu.SemaphoreType.DMA,
            pltpu.SemaphoreType.DMA,
            pltpu.SemaphoreType.DMA,
            pltpu.SemaphoreType.DMA,
            pltpu.SemaphoreType.DMA,
            pltpu.SemaphoreType.DMA,
            pltpu.SemaphoreType.DMA((2,)),
        ],
        compiler_params=_CompilerParams(
            collective_id=0, vmem_limit_bytes=60 * 1024 * 1024
        ),
    )(x, router, W1, W2)
